# baseline (speedup 1.0000x reference)
import functools

import jax
import jax.numpy as jnp
from jax import lax
from jax.experimental import pallas as pl
from jax.experimental.pallas import tpu as pltpu
from jax.experimental.pallas import tpu_sc as plsc

N = 10000
F = 128
E = 320000
NC = 2
NS = 16
NW = NC * NS
CH = 50
SUB = 40
NSUP = 5
NCH = SUB * NSUP
E_PW = NCH * CH
NBUF = 4
ROWS_PT = 640
N_PAD = NS * ROWS_PT


def _mm_body(h_ref, w_ref, o_ref):
    o_ref[...] = jnp.dot(h_ref[...], w_ref[...],
                         preferred_element_type=jnp.float32)


def _matmul(h, W):
    return pl.pallas_call(
        _mm_body,
        grid=(10,),
        in_specs=[
            pl.BlockSpec((N // 10, F), lambda i: (i, 0)),
            pl.BlockSpec((F, F), lambda i: (0, 0)),
        ],
        out_specs=pl.BlockSpec((N // 10, F), lambda i: (i, 0)),
        out_shape=jax.ShapeDtypeStruct((N, F), jnp.float32),
    )(h, W)


_MESH = plsc.VectorSubcoreMesh(core_axis_name="c", subcore_axis_name="s")


@functools.partial(
    pl.kernel,
    out_type=jax.ShapeDtypeStruct((NC, N_PAD, F), jnp.float32),
    mesh=_MESH,
    scratch_types=[
        pltpu.VMEM((SUB, CH), jnp.int32),
        pltpu.VMEM((SUB, CH), jnp.int32),
        *([pltpu.VMEM((CH, F), jnp.float32)] * NBUF),
        pltpu.VMEM_SHARED((N_PAD, F), jnp.float32),
        *([pltpu.SemaphoreType.DMA] * NBUF),
        *([pltpu.SemaphoreType.DMA] * NBUF),
        pltpu.SemaphoreType.DMA,
    ],
)
def _scatter_gather(src_hbm, dst_hbm, hw_hbm, zeros_hbm, out_hbm,
                    src_win, dst_win,
                    rows0, rows1, rows2, rows3, accum,
                    gsem0, gsem1, gsem2, gsem3,
                    ssem0, ssem1, ssem2, ssem3, zsem):
    c = lax.axis_index("c")
    s = lax.axis_index("s")
    wid = c * NS + s
    base = s * ROWS_PT
    bufs = (rows0, rows1, rows2, rows3)
    gsems = (gsem0, gsem1, gsem2, gsem3)
    ssems = (ssem0, ssem1, ssem2, ssem3)

    def wait_gather(b):
        pltpu.make_async_copy(hw_hbm.at[src_win.at[0]],
                              bufs[b], gsems[b]).wait()

    def wait_scatter(b):
        pltpu.make_async_copy(bufs[b], accum.at[dst_win.at[0]],
                              ssems[b]).wait()

    zcp = pltpu.async_copy(zeros_hbm, accum.at[pl.ds(base, ROWS_PT)], zsem)
    pltpu.sync_copy(src_hbm.at[wid, 0], src_win)
    pltpu.sync_copy(dst_hbm.at[wid, 0], dst_win)
    zcp.wait()
    plsc.subcore_barrier()

    for sup in range(NSUP):
        if sup > 0:
            pltpu.sync_copy(src_hbm.at[wid, sup], src_win)
            pltpu.sync_copy(dst_hbm.at[wid, sup], dst_win)

        for b in range(NBUF):
            pltpu.async_copy(hw_hbm.at[src_win.at[b]], bufs[b], gsems[b])

        def chunk_step(it, carry):
            g = it * NBUF
            for b in range(NBUF):
                j = g + b
                wait_gather(b)
                pltpu.async_copy(bufs[b], accum.at[dst_win.at[j]],
                                 ssems[b], add=True)
                wait_scatter(b)
                pltpu.async_copy(hw_hbm.at[src_win.at[j + NBUF]],
                                 bufs[b], gsems[b])
            return carry

        lax.fori_loop(0, (SUB - NBUF) // NBUF, chunk_step, 0)

        for b in range(NBUF):
            j = SUB - NBUF + b
            wait_gather(b)
            pltpu.async_copy(bufs[b], accum.at[dst_win.at[j]],
                             ssems[b], add=True)
        for b in range(NBUF):
            wait_scatter(b)

    plsc.subcore_barrier()

    pltpu.sync_copy(accum.at[pl.ds(base, ROWS_PT)],
                    out_hbm.at[c, pl.ds(base, ROWS_PT)])


def _ep_body(p_ref, b_ref, o_ref):
    o_ref[...] = jnp.maximum(p_ref[0] + p_ref[1] + b_ref[...], 0.0)


def _epilogue(partials, b):
    return pl.pallas_call(
        _ep_body,
        grid=(10,),
        in_specs=[
            pl.BlockSpec((NC, N // 10, F), lambda i: (0, i, 0)),
            pl.BlockSpec((1, F), lambda i: (0, 0)),
        ],
        out_specs=pl.BlockSpec((N // 10, F), lambda i: (i, 0)),
        out_shape=jax.ShapeDtypeStruct((N, F), jnp.float32),
    )(partials, b.reshape(1, F))


def kernel(h, edge_index, W, b):
    e = edge_index.astype(jnp.int32)
    src = e[0].reshape(NW, NSUP, SUB, CH)
    dst = e[1].reshape(NW, NSUP, SUB, CH)
    zeros = jnp.zeros((ROWS_PT, F), jnp.float32)

    hw = _matmul(h, W)
    partials = _scatter_gather(src, dst, hw, zeros)
    return _epilogue(partials, b)


# speedup vs baseline: 1.0769x; 1.0769x over previous
import functools

import jax
import jax.numpy as jnp
from jax import lax
from jax.experimental import pallas as pl
from jax.experimental.pallas import tpu as pltpu
from jax.experimental.pallas import tpu_sc as plsc

N = 10000
F = 128
E = 320000
NC = 2
NS = 16
NW = NC * NS
CH = 125
SUB = 40
NSUP = 2
NCH = SUB * NSUP
E_PW = NCH * CH
NBUF = 2
ROWS_PT = 640
N_PAD = NS * ROWS_PT


_MESH = plsc.VectorSubcoreMesh(core_axis_name="c", subcore_axis_name="s")


@functools.partial(
    pl.kernel,
    out_type=jax.ShapeDtypeStruct((NC, N_PAD, F), jnp.float32),
    mesh=_MESH,
    scratch_types=[
        pltpu.VMEM((SUB, CH), jnp.int32),
        pltpu.VMEM((SUB, CH), jnp.int32),
        pltpu.VMEM((CH, F), jnp.float32),
        pltpu.VMEM((CH, F), jnp.float32),
        pltpu.VMEM_SHARED((N_PAD, F), jnp.float32),
        pltpu.SemaphoreType.DMA,
        pltpu.SemaphoreType.DMA,
        pltpu.SemaphoreType.DMA,
    ],
)
def _scatter_gather(src_hbm, dst_hbm, h_hbm, zeros_hbm, out_hbm,
                    src_win, dst_win, rows0, rows1, accum,
                    gsem0, gsem1, zsem):
    c = lax.axis_index("c")
    s = lax.axis_index("s")
    wid = c * NS + s
    base = s * ROWS_PT
    bufs = (rows0, rows1)
    gsems = (gsem0, gsem1)

    def wait_gather(b):
        pltpu.make_async_copy(h_hbm.at[src_win.at[0]],
                              bufs[b], gsems[b]).wait()

    zcp = pltpu.async_copy(zeros_hbm, accum.at[pl.ds(base, ROWS_PT)], zsem)
    pltpu.sync_copy(src_hbm.at[wid, 0], src_win)
    pltpu.sync_copy(dst_hbm.at[wid, 0], dst_win)
    zcp.wait()
    plsc.subcore_barrier()

    for sup in range(NSUP):
        if sup > 0:
            pltpu.sync_copy(src_hbm.at[wid, sup], src_win)
            pltpu.sync_copy(dst_hbm.at[wid, sup], dst_win)

        for b in range(NBUF):
            pltpu.async_copy(h_hbm.at[src_win.at[b]], bufs[b], gsems[b])

        def chunk_step(it, carry):
            g = it * NBUF
            for b in range(NBUF):
                j = g + b
                wait_gather(b)
                pltpu.sync_copy(bufs[b], accum.at[dst_win.at[j]],
                                add=True)
                pltpu.async_copy(h_hbm.at[src_win.at[j + NBUF]],
                                 bufs[b], gsems[b])
            return carry

        lax.fori_loop(0, (SUB - NBUF) // NBUF, chunk_step, 0)

        for b in range(NBUF):
            j = SUB - NBUF + b
            wait_gather(b)
            pltpu.sync_copy(bufs[b], accum.at[dst_win.at[j]], add=True)

    plsc.subcore_barrier()

    pltpu.sync_copy(accum.at[pl.ds(base, ROWS_PT)],
                    out_hbm.at[c, pl.ds(base, ROWS_PT)])


def _fin_body(p_ref, w_ref, b_ref, o_ref):
    agg = p_ref[0] + p_ref[1]
    o_ref[...] = jnp.maximum(
        jnp.dot(agg, w_ref[...], preferred_element_type=jnp.float32)
        + b_ref[...], 0.0)


def _finalize(partials, W, b):
    return pl.pallas_call(
        _fin_body,
        grid=(10,),
        in_specs=[
            pl.BlockSpec((NC, N // 10, F), lambda i: (0, i, 0)),
            pl.BlockSpec((F, F), lambda i: (0, 0)),
            pl.BlockSpec((1, F), lambda i: (0, 0)),
        ],
        out_specs=pl.BlockSpec((N // 10, F), lambda i: (i, 0)),
        out_shape=jax.ShapeDtypeStruct((N, F), jnp.float32),
    )(partials, W, b.reshape(1, F))


def kernel(h, edge_index, W, b):
    e = edge_index.astype(jnp.int32)
    src = e[0].reshape(NW, NSUP, SUB, CH)
    dst = e[1].reshape(NW, NSUP, SUB, CH)
    zeros = jnp.zeros((ROWS_PT, F), jnp.float32)

    partials = _scatter_gather(src, dst, h, zeros)
    return _finalize(partials, W, b)


# speedup vs baseline: 1.1549x; 1.0724x over previous
import functools

import jax
import jax.numpy as jnp
from jax import lax
from jax.experimental import pallas as pl
from jax.experimental.pallas import tpu as pltpu
from jax.experimental.pallas import tpu_sc as plsc

N = 10000
F = 128
E = 320000
NC = 2
NS = 16
NW = NC * NS
CH = 125
SUB = 40
NSUP = 2
NCH = SUB * NSUP
E_PW = NCH * CH
NBUF = 2
ROWS_PT = 640
N_PAD = NS * ROWS_PT


_MESH = plsc.VectorSubcoreMesh(core_axis_name="c", subcore_axis_name="s")


@functools.partial(
    pl.kernel,
    out_type=jax.ShapeDtypeStruct((NC, N_PAD, F), jnp.float32),
    mesh=_MESH,
    scratch_types=[
        pltpu.VMEM((SUB, CH), jnp.int32),
        pltpu.VMEM((SUB, CH), jnp.int32),
        pltpu.VMEM((CH, F), jnp.float32),
        pltpu.VMEM((CH, F), jnp.float32),
        pltpu.VMEM_SHARED((N_PAD, F), jnp.float32),
        pltpu.SemaphoreType.DMA,
        pltpu.SemaphoreType.DMA,
        pltpu.SemaphoreType.DMA,
    ],
)
def _scatter_gather(edges_hbm, h_hbm, zeros_hbm, out_hbm,
                    src_win, dst_win, rows0, rows1, accum,
                    gsem0, gsem1, zsem):
    c = lax.axis_index("c")
    s = lax.axis_index("s")
    wid = c * NS + s
    base = s * ROWS_PT
    bufs = (rows0, rows1)
    gsems = (gsem0, gsem1)

    def wait_gather(b):
        pltpu.make_async_copy(h_hbm.at[src_win.at[0]],
                              bufs[b], gsems[b]).wait()

    zcp = pltpu.async_copy(zeros_hbm, accum.at[pl.ds(base, ROWS_PT)], zsem)
    pltpu.sync_copy(edges_hbm.at[0, wid, 0], src_win)
    pltpu.sync_copy(edges_hbm.at[1, wid, 0], dst_win)
    zcp.wait()
    plsc.subcore_barrier()

    for sup in range(NSUP):
        if sup > 0:
            pltpu.sync_copy(edges_hbm.at[0, wid, sup], src_win)
            pltpu.sync_copy(edges_hbm.at[1, wid, sup], dst_win)

        for b in range(NBUF):
            pltpu.async_copy(h_hbm.at[src_win.at[b]], bufs[b], gsems[b])

        def chunk_step(it, carry):
            g = it * NBUF
            for b in range(NBUF):
                j = g + b
                wait_gather(b)
                pltpu.sync_copy(bufs[b], accum.at[dst_win.at[j]],
                                add=True)
                pltpu.async_copy(h_hbm.at[src_win.at[j + NBUF]],
                                 bufs[b], gsems[b])
            return carry

        lax.fori_loop(0, (SUB - NBUF) // NBUF, chunk_step, 0)

        for b in range(NBUF):
            j = SUB - NBUF + b
            wait_gather(b)
            pltpu.sync_copy(bufs[b], accum.at[dst_win.at[j]], add=True)

    plsc.subcore_barrier()

    pltpu.sync_copy(accum.at[pl.ds(base, ROWS_PT)],
                    out_hbm.at[c, pl.ds(base, ROWS_PT)])


def _fin_body(p_ref, w_ref, b_ref, o_ref):
    agg = p_ref[0] + p_ref[1]
    o_ref[...] = jnp.maximum(
        jnp.dot(agg, w_ref[...], preferred_element_type=jnp.float32)
        + b_ref[...], 0.0)


def _finalize(partials, W, b):
    return pl.pallas_call(
        _fin_body,
        grid=(10,),
        in_specs=[
            pl.BlockSpec((NC, N // 10, F), lambda i: (0, i, 0)),
            pl.BlockSpec((F, F), lambda i: (0, 0)),
            pl.BlockSpec((1, F), lambda i: (0, 0)),
        ],
        out_specs=pl.BlockSpec((N // 10, F), lambda i: (i, 0)),
        out_shape=jax.ShapeDtypeStruct((N, F), jnp.float32),
    )(partials, W, b.reshape(1, F))


def kernel(h, edge_index, W, b):
    edges = edge_index.astype(jnp.int32).reshape(2, NW, NSUP, SUB, CH)
    zeros = jnp.zeros((ROWS_PT, F), jnp.float32)

    partials = _scatter_gather(edges, h, zeros)
    return _finalize(partials, W, b)
